# baseline (speedup 1.0000x reference)
import functools

import jax
import jax.numpy as jnp
from jax import lax
from jax.experimental import pallas as pl
from jax.experimental.pallas import tpu as pltpu
from jax.experimental.pallas import tpu_sc as plsc

N_NODES = 10000
IN_CH = 256
HID = 256
OUT = 128
N_EDGES = 160000

NC, NS, L = 2, 16, 16
NW = NC * NS

NPAD = 10240
HALF = NPAD // 2
EPAD = 163840
EPW = EPAD // NW
SEG = EPW
RPT = NPAD // NS
APT = HALF // NS

CHUNK = 64
NBUF = 1
GRP = NBUF * CHUNK
ZROW = N_NODES

RBLK = 1024
GROWS = NPAD // RBLK

_mesh = plsc.VectorSubcoreMesh(core_axis_name="c", subcore_axis_name="s")


@functools.partial(
    pl.kernel,
    out_type=(
        jax.ShapeDtypeStruct((NC, NPAD), jnp.float32),
        jax.ShapeDtypeStruct((NC, NW, SEG), jnp.int32),
        jax.ShapeDtypeStruct((NC, NW, SEG), jnp.int32),
        jax.ShapeDtypeStruct((NC, NW, L), jnp.int32),
    ),
    mesh=_mesh,
    compiler_params=pltpu.CompilerParams(needs_layout_passes=False),
    scratch_types=[
        pltpu.VMEM((EPW,), jnp.int32),
        pltpu.VMEM((EPW,), jnp.int32),
        pltpu.VMEM((SEG,), jnp.int32),
        pltpu.VMEM((SEG,), jnp.int32),
        pltpu.VMEM((NPAD,), jnp.float32),
        pltpu.VMEM((NS, RPT), jnp.float32),
        pltpu.VMEM((RPT,), jnp.float32),
        pltpu.VMEM((L,), jnp.int32),
        pltpu.VMEM_SHARED((NS, NPAD), jnp.float32),
    ],
)
def _deg_part_kernel(src_hbm, dst_hbm, deg_hbm, psrc_hbm, pdst_hbm, cnt_hbm,
                     sbuf, dbuf, csrc, cdst, hist, red2, red, cvec, sh):
    c = lax.axis_index("c")
    s = lax.axis_index("s")
    g = c * NS + s

    pltpu.sync_copy(src_hbm.at[pl.ds(g * EPW, EPW)], sbuf)
    pltpu.sync_copy(dst_hbm.at[pl.ds(g * EPW, EPW)], dbuf)

    def zero(i, _):
        hist[pl.ds(i * L, L)] = jnp.zeros((L,), jnp.float32)
        return 0

    lax.fori_loop(0, NPAD // L, zero, 0)
    ones = jnp.ones((L,), jnp.float32)

    def scat(i, _):
        idx = dbuf[pl.ds(i * L, L)]
        plsc.addupdate_scatter(hist, [idx], ones)
        return 0

    lax.fori_loop(0, EPW // L, scat, 0)

    pltpu.sync_copy(hist, sh.at[s])

    for bkt in range(NC):
        lo = bkt * HALF

        def prefill(i, _):
            csrc[pl.ds(i * L, L)] = jnp.full((L,), ZROW, jnp.int32)
            cdst[pl.ds(i * L, L)] = jnp.zeros((L,), jnp.int32)
            return 0

        lax.fori_loop(0, SEG // L, prefill, 0)

        def compact(i, off):
            sv = sbuf[pl.ds(i * L, L)]
            dv = dbuf[pl.ds(i * L, L)]
            m = (dv >= lo) & (dv < lo + HALF)
            mi = m.astype(jnp.int32)
            pos = off + jnp.cumsum(mi) - 1
            plsc.store_scatter(csrc, [pos], sv, mask=m)
            plsc.store_scatter(cdst, [pos], dv - lo, mask=m)
            return off + jnp.sum(mi)

        cn = lax.fori_loop(0, EPW // L, compact, jnp.int32(0))
        ngrp = (cn + (GRP - 1)) // GRP
        cvec[...] = jnp.full((L,), 1, jnp.int32) * ngrp
        pltpu.sync_copy(csrc, psrc_hbm.at[bkt, g])
        pltpu.sync_copy(cdst, pdst_hbm.at[bkt, g])
        pltpu.sync_copy(cvec, cnt_hbm.at[bkt, g])

    plsc.subcore_barrier()
    base = s * RPT
    pltpu.sync_copy(sh.at[:, pl.ds(base, RPT)], red2)

    def reduce(i, _):
        a = red2[0, pl.ds(i * L, L)]
        for t in range(1, NS):
            a = a + red2[t, pl.ds(i * L, L)]
        red[pl.ds(i * L, L)] = a
        return 0

    lax.fori_loop(0, RPT // L, reduce, 0)
    pltpu.sync_copy(red, deg_hbm.at[c, pl.ds(base, RPT)])


@functools.partial(
    pl.kernel,
    out_type=jax.ShapeDtypeStruct((NPAD, 2, 128), jnp.float32),
    mesh=_mesh,
    scratch_types=(
        [pltpu.VMEM((CHUNK,), jnp.int32)] * NBUF
        + [pltpu.VMEM((CHUNK,), jnp.int32)] * NBUF
        + [pltpu.VMEM((CHUNK, 2, 128), jnp.float32)] * NBUF
        + [pltpu.VMEM((L,), jnp.int32)] * 2
        + [pltpu.VMEM_SHARED((HALF, 2, 128), jnp.float32)]
        + [pltpu.SemaphoreType.DMA] * (2 * NBUF)
    ),
)
def _spmm_kernel(hs_hbm, psrc_hbm, pdst_hbm, cnt_hbm, z_hbm, out_hbm, *refs):
    isrc = refs[0:NBUF]
    idst = refs[NBUF:2 * NBUF]
    rows = refs[2 * NBUF:3 * NBUF]
    cva, cvb = refs[3 * NBUF], refs[3 * NBUF + 1]
    acc = refs[2 + 3 * NBUF]
    gsem = refs[3 + 3 * NBUF:3 + 4 * NBUF]
    ssem = refs[3 + 4 * NBUF:3 + 5 * NBUF]

    c = lax.axis_index("c")
    s = lax.axis_index("s")
    abase = s * APT
    pltpu.sync_copy(z_hbm.at[pl.ds(abase, APT)], acc.at[pl.ds(abase, APT)])
    pltpu.sync_copy(cnt_hbm.at[c, 2 * s], cva)
    pltpu.sync_copy(cnt_hbm.at[c, 2 * s + 1], cvb)
    plsc.subcore_barrier()

    for seg in range(2):
        g = 2 * s + seg
        n = (cva if seg == 0 else cvb)[...][0]

        def ring(k, _):
            idd = []
            for b in range(NBUF):
                off = (k * NBUF + b) * CHUNK

                @pl.when(k > 0)
                def _drain():
                    pltpu.make_async_copy(z_hbm.at[pl.ds(0, CHUNK)],
                                          rows[b], ssem[b]).wait()

                d1 = pltpu.async_copy(psrc_hbm.at[c, g, pl.ds(off, CHUNK)],
                                      isrc[b], gsem[b])
                d2 = pltpu.async_copy(pdst_hbm.at[c, g, pl.ds(off, CHUNK)],
                                      idst[b], gsem[b])
                idd.append((d1, d2))
            gd = []
            for b in range(NBUF):
                idd[b][0].wait()
                idd[b][1].wait()
                gd.append(pltpu.async_copy(hs_hbm.at[isrc[b]], rows[b], gsem[b]))
            for b in range(NBUF):
                gd[b].wait()
                pltpu.async_copy(rows[b], acc.at[idst[b]], ssem[b], add=True)
            return 0

        lax.fori_loop(0, n, ring, 0)
        for b in range(NBUF):

            @pl.when(n > 0)
            def _fdrain():
                pltpu.make_async_copy(z_hbm.at[pl.ds(0, CHUNK)],
                                      rows[b], ssem[b]).wait()

    plsc.subcore_barrier()
    pltpu.sync_copy(acc.at[pl.ds(abase, APT)],
                    out_hbm.at[pl.ds(c * HALF + abase, APT)])


def _dis(d0_ref, d1_ref):
    deg = d0_ref[0, 0, :] + d1_ref[0, 0, :] + 1.0
    return lax.rsqrt(deg)


def _rowmask(j, x):
    rid = j * RBLK + lax.broadcasted_iota(jnp.int32, (RBLK, 1), 0)
    return jnp.where(rid < N_NODES, x, 0.0)


def _stage1_body(x_ref, w_ref, d0_ref, d1_ref, o_ref):
    j = pl.program_id(0)
    dis = _dis(d0_ref, d1_ref)
    h = jnp.dot(x_ref[...], w_ref[...], preferred_element_type=jnp.float32)
    o_ref[...] = _rowmask(j, h * dis[:, None])


_stage1 = pl.pallas_call(
    _stage1_body,
    grid=(GROWS,),
    in_specs=[
        pl.BlockSpec((RBLK, IN_CH), lambda j: (j, 0)),
        pl.BlockSpec((IN_CH, HID), lambda j: (0, 0)),
        pl.BlockSpec((1, 1, RBLK), lambda j: (j, 0, 0)),
        pl.BlockSpec((1, 1, RBLK), lambda j: (j, 0, 0)),
    ],
    out_specs=pl.BlockSpec((RBLK, HID), lambda j: (j, 0)),
    out_shape=jax.ShapeDtypeStruct((NPAD, HID), jnp.float32),
)


def _stage2_body(a_ref, h_ref, d0_ref, d1_ref, b_ref, w_ref, o_ref):
    j = pl.program_id(0)
    dis = _dis(d0_ref, d1_ref)
    h = (a_ref[...] + h_ref[...]) * dis[:, None] + b_ref[0, :][None, :]
    h = jnp.maximum(h, 0.0)
    hc = jnp.dot(h, w_ref[...], preferred_element_type=jnp.float32)
    o_ref[...] = _rowmask(j, hc * dis[:, None])


_stage2 = pl.pallas_call(
    _stage2_body,
    grid=(GROWS,),
    in_specs=[
        pl.BlockSpec((RBLK, HID), lambda j: (j, 0)),
        pl.BlockSpec((RBLK, HID), lambda j: (j, 0)),
        pl.BlockSpec((1, 1, RBLK), lambda j: (j, 0, 0)),
        pl.BlockSpec((1, 1, RBLK), lambda j: (j, 0, 0)),
        pl.BlockSpec((1, HID), lambda j: (0, 0)),
        pl.BlockSpec((HID, HID), lambda j: (0, 0)),
    ],
    out_specs=pl.BlockSpec((RBLK, HID), lambda j: (j, 0)),
    out_shape=jax.ShapeDtypeStruct((NPAD, HID), jnp.float32),
)


def _stage3_body(a0, a1, h0, h1, d0_ref, d1_ref, bmu_ref, bls_ref,
                 mu_ref, ls_ref):
    dis = _dis(d0_ref, d1_ref)
    mu_ref[...] = (a0[...] + h0[...]) * dis[:, None] + bmu_ref[0, :][None, :]
    ls_ref[...] = (a1[...] + h1[...]) * dis[:, None] + bls_ref[0, :][None, :]


_stage3 = pl.pallas_call(
    _stage3_body,
    grid=(GROWS,),
    in_specs=[
        pl.BlockSpec((RBLK, OUT), lambda j: (j, 0)),
        pl.BlockSpec((RBLK, OUT), lambda j: (j, 1)),
        pl.BlockSpec((RBLK, OUT), lambda j: (j, 0)),
        pl.BlockSpec((RBLK, OUT), lambda j: (j, 1)),
        pl.BlockSpec((1, 1, RBLK), lambda j: (j, 0, 0)),
        pl.BlockSpec((1, 1, RBLK), lambda j: (j, 0, 0)),
        pl.BlockSpec((1, OUT), lambda j: (0, 0)),
        pl.BlockSpec((1, OUT), lambda j: (0, 0)),
    ],
    out_specs=[
        pl.BlockSpec((RBLK, OUT), lambda j: (j, 0)),
        pl.BlockSpec((RBLK, OUT), lambda j: (j, 0)),
    ],
    out_shape=[
        jax.ShapeDtypeStruct((NPAD, OUT), jnp.float32),
        jax.ShapeDtypeStruct((NPAD, OUT), jnp.float32),
    ],
)


def kernel(x, edge_index, W1, b1, Wmu, bmu, Wls, bls):
    src = edge_index[0].astype(jnp.int32)
    dst = edge_index[1].astype(jnp.int32)
    pad = EPAD - N_EDGES
    srcp = jnp.concatenate([src, jnp.full((pad,), ZROW, jnp.int32)])
    dstp = jnp.concatenate([dst, jnp.full((pad,), N_NODES, jnp.int32)])
    xp = jnp.pad(x, ((0, NPAD - N_NODES), (0, 0)))
    z = jnp.zeros((HALF, 2, 128), jnp.float32)
    Wcat = jnp.concatenate([Wmu, Wls], axis=1)

    degp, psrc, pdst, cnt = _deg_part_kernel(srcp, dstp)
    d0 = degp[0].reshape(GROWS, 1, RBLK)
    d1 = degp[1].reshape(GROWS, 1, RBLK)

    hs1 = _stage1(xp, W1, d0, d1)
    agg1 = _spmm_kernel(hs1.reshape(NPAD, 2, 128), psrc, pdst, cnt,
                        z).reshape(NPAD, HID)
    hs2 = _stage2(agg1, hs1, d0, d1, b1.reshape(1, HID), Wcat)
    agg2 = _spmm_kernel(hs2.reshape(NPAD, 2, 128), psrc, pdst, cnt,
                        z).reshape(NPAD, HID)
    mu, ls = _stage3(agg2, agg2, hs2, hs2, d0, d1,
                     bmu.reshape(1, OUT), bls.reshape(1, OUT))
    return mu[:N_NODES], ls[:N_NODES]


# speedup vs baseline: 1.2182x; 1.2182x over previous
import functools

import jax
import jax.numpy as jnp
from jax import lax
from jax.experimental import pallas as pl
from jax.experimental.pallas import tpu as pltpu
from jax.experimental.pallas import tpu_sc as plsc

N_NODES = 10000
IN_CH = 256
HID = 256
OUT = 128
N_EDGES = 160000

NC, NS, L = 2, 16, 16
NW = NC * NS

NPAD = 10240
RPT = NPAD // NS
EPAD = 163840
CHUNK = 128
EPT = EPAD // NS
EPW = EPAD // NW
NBUF = 2
NCHUNK = EPT // CHUNK
OUTER = NCHUNK // NBUF

RBLK = 1024
GROWS = NPAD // RBLK

_mesh = plsc.VectorSubcoreMesh(core_axis_name="c", subcore_axis_name="s")


@functools.partial(
    pl.kernel,
    out_type=jax.ShapeDtypeStruct((NC, NPAD), jnp.float32),
    mesh=_mesh,
    compiler_params=pltpu.CompilerParams(needs_layout_passes=False),
    scratch_types=[
        pltpu.VMEM((EPW,), jnp.int32),
        pltpu.VMEM((NPAD,), jnp.float32),
        pltpu.VMEM((NS, RPT), jnp.float32),
        pltpu.VMEM((RPT,), jnp.float32),
        pltpu.VMEM_SHARED((NS, NPAD), jnp.float32),
    ],
)
def _deg_kernel(dst_hbm, out_hbm, dbuf, hist, red2, red, sh):
    c = lax.axis_index("c")
    s = lax.axis_index("s")
    g = c * NS + s

    def zero(i, _):
        hist[pl.ds(i * L, L)] = jnp.zeros((L,), jnp.float32)
        return 0

    lax.fori_loop(0, NPAD // L, zero, 0)

    pltpu.sync_copy(dst_hbm.at[pl.ds(g * EPW, EPW)], dbuf)
    ones = jnp.ones((L,), jnp.float32)

    def scat(i, _):
        idx = dbuf[pl.ds(i * L, L)]
        plsc.addupdate_scatter(hist, [idx], ones)
        return 0

    lax.fori_loop(0, EPW // L, scat, 0)

    pltpu.sync_copy(hist, sh.at[s])
    plsc.subcore_barrier()

    base = s * RPT
    pltpu.sync_copy(sh.at[:, pl.ds(base, RPT)], red2)

    def reduce(i, _):
        a = red2[0, pl.ds(i * L, L)]
        for t in range(1, NS):
            a = a + red2[t, pl.ds(i * L, L)]
        red[pl.ds(i * L, L)] = a
        return 0

    lax.fori_loop(0, RPT // L, reduce, 0)
    pltpu.sync_copy(red, out_hbm.at[c, pl.ds(base, RPT)])


@functools.partial(
    pl.kernel,
    out_type=jax.ShapeDtypeStruct((NC * NPAD, OUT), jnp.float32),
    mesh=_mesh,
    scratch_types=(
        [pltpu.VMEM((CHUNK,), jnp.int32)]
        + [pltpu.VMEM((CHUNK,), jnp.int32)]
        + [pltpu.VMEM((CHUNK, OUT), jnp.float32)]
        + [pltpu.VMEM_SHARED((NPAD, OUT), jnp.float32)]
        + [pltpu.SemaphoreType.DMA]
    ),
)
def _spmm_kernel(hs_hbm, src2_hbm, dst_hbm, z_hbm, out_hbm, *refs):
    isrc, idst, rows, acc, sem = refs

    c = lax.axis_index("c")
    s = lax.axis_index("s")
    base = s * RPT
    pltpu.sync_copy(z_hbm.at[pl.ds(base, RPT), :], acc.at[pl.ds(base, RPT), :])
    plsc.subcore_barrier()

    e0 = s * EPT

    def step(j, _):
        off = e0 + j * CHUNK
        pltpu.sync_copy(src2_hbm.at[c, pl.ds(off, CHUNK)], isrc)
        pltpu.sync_copy(dst_hbm.at[pl.ds(off, CHUNK)], idst)
        pltpu.async_copy(hs_hbm.at[isrc], rows, sem).wait()
        pltpu.sync_copy(rows, acc.at[idst], add=True)
        return 0

    lax.fori_loop(0, NCHUNK, step, 0)
    plsc.subcore_barrier()
    pltpu.sync_copy(acc.at[pl.ds(base, RPT), :],
                    out_hbm.at[pl.ds(c * NPAD + base, RPT), :])


def _dis(d0_ref, d1_ref):
    deg = d0_ref[0, 0, :] + d1_ref[0, 0, :] + 1.0
    return lax.rsqrt(deg)


def _stage1_body(x_ref, w_ref, d0_ref, d1_ref, o_ref):
    dis = _dis(d0_ref, d1_ref)
    h = jnp.dot(x_ref[...], w_ref[...], preferred_element_type=jnp.float32)
    o_ref[...] = h * dis[:, None]


_stage1 = pl.pallas_call(
    _stage1_body,
    grid=(NC, GROWS),
    in_specs=[
        pl.BlockSpec((RBLK, IN_CH), lambda c, j: (j, 0)),
        pl.BlockSpec((IN_CH, OUT), lambda c, j: (0, c)),
        pl.BlockSpec((1, 1, RBLK), lambda c, j: (j, 0, 0)),
        pl.BlockSpec((1, 1, RBLK), lambda c, j: (j, 0, 0)),
    ],
    out_specs=pl.BlockSpec((RBLK, OUT), lambda c, j: (c * GROWS + j, 0)),
    out_shape=jax.ShapeDtypeStruct((NC * NPAD, OUT), jnp.float32),
)


def _stage2_body(a0_ref, a1_ref, h0_ref, h1_ref, d0_ref, d1_ref, b_ref,
                 w_ref, o_ref):
    dis = _dis(d0_ref, d1_ref)
    pre0 = (a0_ref[...] + h0_ref[...]) * dis[:, None]
    pre1 = (a1_ref[...] + h1_ref[...]) * dis[:, None]
    h = jnp.concatenate([pre0, pre1], axis=1) + b_ref[0, :][None, :]
    h = jnp.maximum(h, 0.0)
    hc = jnp.dot(h, w_ref[...], preferred_element_type=jnp.float32)
    o_ref[...] = hc * dis[:, None]


_stage2 = pl.pallas_call(
    _stage2_body,
    grid=(NC, GROWS),
    in_specs=[
        pl.BlockSpec((RBLK, OUT), lambda c, j: (j, 0)),
        pl.BlockSpec((RBLK, OUT), lambda c, j: (GROWS + j, 0)),
        pl.BlockSpec((RBLK, OUT), lambda c, j: (j, 0)),
        pl.BlockSpec((RBLK, OUT), lambda c, j: (GROWS + j, 0)),
        pl.BlockSpec((1, 1, RBLK), lambda c, j: (j, 0, 0)),
        pl.BlockSpec((1, 1, RBLK), lambda c, j: (j, 0, 0)),
        pl.BlockSpec((1, HID), lambda c, j: (0, 0)),
        pl.BlockSpec((HID, OUT), lambda c, j: (0, c)),
    ],
    out_specs=pl.BlockSpec((RBLK, OUT), lambda c, j: (c * GROWS + j, 0)),
    out_shape=jax.ShapeDtypeStruct((NC * NPAD, OUT), jnp.float32),
)


def _stage3_body(a0_ref, a1_ref, h0_ref, h1_ref, d0_ref, d1_ref,
                 bmu_ref, bls_ref, mu_ref, ls_ref):
    dis = _dis(d0_ref, d1_ref)
    mu_ref[...] = (a0_ref[...] + h0_ref[...]) * dis[:, None] + bmu_ref[0, :][None, :]
    ls_ref[...] = (a1_ref[...] + h1_ref[...]) * dis[:, None] + bls_ref[0, :][None, :]


_stage3 = pl.pallas_call(
    _stage3_body,
    grid=(GROWS,),
    in_specs=[
        pl.BlockSpec((RBLK, OUT), lambda j: (j, 0)),
        pl.BlockSpec((RBLK, OUT), lambda j: (GROWS + j, 0)),
        pl.BlockSpec((RBLK, OUT), lambda j: (j, 0)),
        pl.BlockSpec((RBLK, OUT), lambda j: (GROWS + j, 0)),
        pl.BlockSpec((1, 1, RBLK), lambda j: (j, 0, 0)),
        pl.BlockSpec((1, 1, RBLK), lambda j: (j, 0, 0)),
        pl.BlockSpec((1, OUT), lambda j: (0, 0)),
        pl.BlockSpec((1, OUT), lambda j: (0, 0)),
    ],
    out_specs=[
        pl.BlockSpec((RBLK, OUT), lambda j: (j, 0)),
        pl.BlockSpec((RBLK, OUT), lambda j: (j, 0)),
    ],
    out_shape=[
        jax.ShapeDtypeStruct((NPAD, OUT), jnp.float32),
        jax.ShapeDtypeStruct((NPAD, OUT), jnp.float32),
    ],
)


def kernel(x, edge_index, W1, b1, Wmu, bmu, Wls, bls):
    src = edge_index[0].astype(jnp.int32)
    dst = edge_index[1].astype(jnp.int32)
    pad = EPAD - N_EDGES
    srcp = jnp.concatenate([src, jnp.zeros((pad,), jnp.int32)])
    dstp = jnp.concatenate([dst, jnp.full((pad,), N_NODES, jnp.int32)])
    src2 = jnp.stack([srcp, srcp + NPAD])
    xp = jnp.pad(x, ((0, NPAD - N_NODES), (0, 0)))
    z = jnp.zeros((NPAD, OUT), jnp.float32)
    Wcat = jnp.concatenate([Wmu, Wls], axis=1)

    degp = _deg_kernel(dstp)
    d0 = degp[0].reshape(GROWS, 1, RBLK)
    d1 = degp[1].reshape(GROWS, 1, RBLK)

    hs1 = _stage1(xp, W1, d0, d1)
    agg1 = _spmm_kernel(hs1, src2, dstp, z)
    hs2 = _stage2(agg1, agg1, hs1, hs1, d0, d1, b1.reshape(1, HID), Wcat)
    agg2 = _spmm_kernel(hs2, src2, dstp, z)
    mu, ls = _stage3(agg2, agg2, hs2, hs2, d0, d1,
                     bmu.reshape(1, OUT), bls.reshape(1, OUT))
    return mu[:N_NODES], ls[:N_NODES]


# speedup vs baseline: 1.4126x; 1.1596x over previous
import functools

import jax
import jax.numpy as jnp
from jax import lax
from jax.experimental import pallas as pl
from jax.experimental.pallas import tpu as pltpu
from jax.experimental.pallas import tpu_sc as plsc

N_NODES = 10000
IN_CH = 256
HID = 256
OUT = 128
N_EDGES = 160000

NC, NS, L = 2, 16, 16
NW = NC * NS

NPAD = 10240
RPT = NPAD // NS
EPAD = 163840
CHUNK = 128
EPT = EPAD // NS
EPW = EPAD // NW
NBUF = 2
NCHUNK = EPT // CHUNK
OUTER = NCHUNK // NBUF

RBLK = 1024
GROWS = NPAD // RBLK

_mesh = plsc.VectorSubcoreMesh(core_axis_name="c", subcore_axis_name="s")


@functools.partial(
    pl.kernel,
    out_type=jax.ShapeDtypeStruct((NC, NPAD), jnp.float32),
    mesh=_mesh,
    compiler_params=pltpu.CompilerParams(needs_layout_passes=False),
    scratch_types=[
        pltpu.VMEM((EPW,), jnp.int32),
        pltpu.VMEM((NPAD,), jnp.float32),
        pltpu.VMEM((NS, RPT), jnp.float32),
        pltpu.VMEM((RPT,), jnp.float32),
        pltpu.VMEM_SHARED((NS, NPAD), jnp.float32),
    ],
)
def _deg_kernel(dst_hbm, out_hbm, dbuf, hist, red2, red, sh):
    c = lax.axis_index("c")
    s = lax.axis_index("s")
    g = c * NS + s

    def zero(i, _):
        hist[pl.ds(i * L, L)] = jnp.zeros((L,), jnp.float32)
        return 0

    lax.fori_loop(0, NPAD // L, zero, 0)

    pltpu.sync_copy(dst_hbm.at[pl.ds(g * EPW, EPW)], dbuf)
    ones = jnp.ones((L,), jnp.float32)

    def scat(i, _):
        idx = dbuf[pl.ds(i * L, L)]
        plsc.addupdate_scatter(hist, [idx], ones)
        return 0

    lax.fori_loop(0, EPW // L, scat, 0)

    pltpu.sync_copy(hist, sh.at[s])
    plsc.subcore_barrier()

    base = s * RPT
    pltpu.sync_copy(sh.at[:, pl.ds(base, RPT)], red2)

    def reduce(i, _):
        a = red2[0, pl.ds(i * L, L)]
        for t in range(1, NS):
            a = a + red2[t, pl.ds(i * L, L)]
        red[pl.ds(i * L, L)] = a
        return 0

    lax.fori_loop(0, RPT // L, reduce, 0)
    pltpu.sync_copy(red, out_hbm.at[c, pl.ds(base, RPT)])


@functools.partial(
    pl.kernel,
    out_type=jax.ShapeDtypeStruct((NC * NPAD, OUT), jnp.float32),
    mesh=_mesh,
    scratch_types=(
        [pltpu.VMEM((CHUNK,), jnp.int32)] * 2
        + [pltpu.VMEM((CHUNK,), jnp.int32)] * 2
        + [pltpu.VMEM((CHUNK, OUT), jnp.float32)]
        + [pltpu.VMEM_SHARED((NPAD, OUT), jnp.float32)]
        + [pltpu.SemaphoreType.DMA] * 3
    ),
)
def _spmm_kernel(hs_hbm, src2_hbm, dst_hbm, z_hbm, out_hbm, *refs):
    isrc = refs[0:2]
    idst = refs[2:4]
    rows, acc, sem = refs[4], refs[5], refs[6]
    isem = refs[7:9]

    c = lax.axis_index("c")
    s = lax.axis_index("s")
    base = s * RPT
    pltpu.sync_copy(z_hbm.at[pl.ds(base, RPT), :], acc.at[pl.ds(base, RPT), :])
    plsc.subcore_barrier()

    e0 = s * EPT

    def fire_idx(j, b):
        off = e0 + j * CHUNK
        pltpu.async_copy(src2_hbm.at[c, pl.ds(off, CHUNK)], isrc[b], isem[b])
        pltpu.async_copy(dst_hbm.at[pl.ds(off, CHUNK)], idst[b], isem[b])

    for b in range(2):
        fire_idx(b, b)

    def step(g, _):
        for b in range(2):
            j = 2 * g + b
            pltpu.make_async_copy(src2_hbm.at[c, pl.ds(0, CHUNK)],
                                  isrc[b], isem[b]).wait()
            pltpu.make_async_copy(dst_hbm.at[pl.ds(0, CHUNK)],
                                  idst[b], isem[b]).wait()
            pltpu.async_copy(hs_hbm.at[isrc[b]], rows, sem).wait()

            @pl.when(j + 2 < NCHUNK)
            def _prefetch():
                fire_idx(j + 2, b)

            pltpu.sync_copy(rows, acc.at[idst[b]], add=True)
        return 0

    lax.fori_loop(0, NCHUNK // 2, step, 0)
    plsc.subcore_barrier()
    pltpu.sync_copy(acc.at[pl.ds(base, RPT), :],
                    out_hbm.at[pl.ds(c * NPAD + base, RPT), :])


def _dis(d0_ref, d1_ref):
    deg = d0_ref[0, 0, :] + d1_ref[0, 0, :] + 1.0
    return lax.rsqrt(deg)


def _stage1_body(x_ref, w_ref, d0_ref, d1_ref, o_ref):
    dis = _dis(d0_ref, d1_ref)
    h = jnp.dot(x_ref[...], w_ref[...], preferred_element_type=jnp.float32)
    o_ref[...] = h * dis[:, None]


_stage1 = pl.pallas_call(
    _stage1_body,
    grid=(NC, GROWS),
    in_specs=[
        pl.BlockSpec((RBLK, IN_CH), lambda c, j: (j, 0)),
        pl.BlockSpec((IN_CH, OUT), lambda c, j: (0, c)),
        pl.BlockSpec((1, 1, RBLK), lambda c, j: (j, 0, 0)),
        pl.BlockSpec((1, 1, RBLK), lambda c, j: (j, 0, 0)),
    ],
    out_specs=pl.BlockSpec((RBLK, OUT), lambda c, j: (c * GROWS + j, 0)),
    out_shape=jax.ShapeDtypeStruct((NC * NPAD, OUT), jnp.float32),
)


def _stage2_body(a0_ref, a1_ref, h0_ref, h1_ref, d0_ref, d1_ref, b_ref,
                 w_ref, o_ref):
    dis = _dis(d0_ref, d1_ref)
    pre0 = (a0_ref[...] + h0_ref[...]) * dis[:, None]
    pre1 = (a1_ref[...] + h1_ref[...]) * dis[:, None]
    h = jnp.concatenate([pre0, pre1], axis=1) + b_ref[0, :][None, :]
    h = jnp.maximum(h, 0.0)
    hc = jnp.dot(h, w_ref[...], preferred_element_type=jnp.float32)
    o_ref[...] = hc * dis[:, None]


_stage2 = pl.pallas_call(
    _stage2_body,
    grid=(NC, GROWS),
    in_specs=[
        pl.BlockSpec((RBLK, OUT), lambda c, j: (j, 0)),
        pl.BlockSpec((RBLK, OUT), lambda c, j: (GROWS + j, 0)),
        pl.BlockSpec((RBLK, OUT), lambda c, j: (j, 0)),
        pl.BlockSpec((RBLK, OUT), lambda c, j: (GROWS + j, 0)),
        pl.BlockSpec((1, 1, RBLK), lambda c, j: (j, 0, 0)),
        pl.BlockSpec((1, 1, RBLK), lambda c, j: (j, 0, 0)),
        pl.BlockSpec((1, HID), lambda c, j: (0, 0)),
        pl.BlockSpec((HID, OUT), lambda c, j: (0, c)),
    ],
    out_specs=pl.BlockSpec((RBLK, OUT), lambda c, j: (c * GROWS + j, 0)),
    out_shape=jax.ShapeDtypeStruct((NC * NPAD, OUT), jnp.float32),
)


def _stage3_body(a0_ref, a1_ref, h0_ref, h1_ref, d0_ref, d1_ref,
                 bmu_ref, bls_ref, mu_ref, ls_ref):
    dis = _dis(d0_ref, d1_ref)
    mu_ref[...] = (a0_ref[...] + h0_ref[...]) * dis[:, None] + bmu_ref[0, :][None, :]
    ls_ref[...] = (a1_ref[...] + h1_ref[...]) * dis[:, None] + bls_ref[0, :][None, :]


_stage3 = pl.pallas_call(
    _stage3_body,
    grid=(GROWS,),
    in_specs=[
        pl.BlockSpec((RBLK, OUT), lambda j: (j, 0)),
        pl.BlockSpec((RBLK, OUT), lambda j: (GROWS + j, 0)),
        pl.BlockSpec((RBLK, OUT), lambda j: (j, 0)),
        pl.BlockSpec((RBLK, OUT), lambda j: (GROWS + j, 0)),
        pl.BlockSpec((1, 1, RBLK), lambda j: (j, 0, 0)),
        pl.BlockSpec((1, 1, RBLK), lambda j: (j, 0, 0)),
        pl.BlockSpec((1, OUT), lambda j: (0, 0)),
        pl.BlockSpec((1, OUT), lambda j: (0, 0)),
    ],
    out_specs=[
        pl.BlockSpec((RBLK, OUT), lambda j: (j, 0)),
        pl.BlockSpec((RBLK, OUT), lambda j: (j, 0)),
    ],
    out_shape=[
        jax.ShapeDtypeStruct((NPAD, OUT), jnp.float32),
        jax.ShapeDtypeStruct((NPAD, OUT), jnp.float32),
    ],
)


def kernel(x, edge_index, W1, b1, Wmu, bmu, Wls, bls):
    src = edge_index[0].astype(jnp.int32)
    dst = edge_index[1].astype(jnp.int32)
    pad = EPAD - N_EDGES
    srcp = jnp.concatenate([src, jnp.zeros((pad,), jnp.int32)])
    dstp = jnp.concatenate([dst, jnp.full((pad,), N_NODES, jnp.int32)])
    src2 = jnp.stack([srcp, srcp + NPAD])
    xp = jnp.pad(x, ((0, NPAD - N_NODES), (0, 0)))
    z = jnp.zeros((NPAD, OUT), jnp.float32)
    Wcat = jnp.concatenate([Wmu, Wls], axis=1)

    degp = _deg_kernel(dstp)
    d0 = degp[0].reshape(GROWS, 1, RBLK)
    d1 = degp[1].reshape(GROWS, 1, RBLK)

    hs1 = _stage1(xp, W1, d0, d1)
    agg1 = _spmm_kernel(hs1, src2, dstp, z)
    hs2 = _stage2(agg1, agg1, hs1, hs1, d0, d1, b1.reshape(1, HID), Wcat)
    agg2 = _spmm_kernel(hs2, src2, dstp, z)
    mu, ls = _stage3(agg2, agg2, hs2, hs2, d0, d1,
                     bmu.reshape(1, OUT), bls.reshape(1, OUT))
    return mu[:N_NODES], ls[:N_NODES]


# speedup vs baseline: 1.4153x; 1.0019x over previous
import functools

import jax
import jax.numpy as jnp
from jax import lax
from jax.experimental import pallas as pl
from jax.experimental.pallas import tpu as pltpu
from jax.experimental.pallas import tpu_sc as plsc

N_NODES = 10000
IN_CH = 256
HID = 256
OUT = 128
N_EDGES = 160000

NC, NS, L = 2, 16, 16
NW = NC * NS

NPAD = 10240
RPT = NPAD // NS
EPAD = 163840
CHUNK = 128
EPT = EPAD // NS
EPW = EPAD // NW
NCHUNK = EPT // CHUNK

RBLK = 1024
GROWS = NPAD // RBLK

_mesh = plsc.VectorSubcoreMesh(core_axis_name="c", subcore_axis_name="s")


@functools.partial(
    pl.kernel,
    out_type=jax.ShapeDtypeStruct((NC, NPAD), jnp.float32),
    mesh=_mesh,
    compiler_params=pltpu.CompilerParams(needs_layout_passes=False),
    scratch_types=[
        pltpu.VMEM((EPW,), jnp.int32),
        pltpu.VMEM((NPAD,), jnp.float32),
        pltpu.VMEM((NS, RPT), jnp.float32),
        pltpu.VMEM((RPT,), jnp.float32),
        pltpu.VMEM_SHARED((NS, NPAD), jnp.float32),
    ],
)
def _deg_kernel(dst_hbm, out_hbm, dbuf, hist, red2, red, sh):
    c = lax.axis_index("c")
    s = lax.axis_index("s")
    g = c * NS + s

    def zero(i, _):
        hist[pl.ds(i * L, L)] = jnp.zeros((L,), jnp.float32)
        return 0

    lax.fori_loop(0, NPAD // L, zero, 0)

    pltpu.sync_copy(dst_hbm.at[pl.ds(g * EPW, EPW)], dbuf)
    ones = jnp.ones((L,), jnp.float32)

    def scat(i, _):
        idx = dbuf[pl.ds(i * L, L)]
        plsc.addupdate_scatter(hist, [idx], ones)
        return 0

    lax.fori_loop(0, EPW // L, scat, 0)

    pltpu.sync_copy(hist, sh.at[s])
    plsc.subcore_barrier()

    base = s * RPT
    pltpu.sync_copy(sh.at[:, pl.ds(base, RPT)], red2)

    def reduce(i, _):
        a = red2[0, pl.ds(i * L, L)]
        for t in range(1, NS):
            a = a + red2[t, pl.ds(i * L, L)]
        red[pl.ds(i * L, L)] = a
        return 0

    lax.fori_loop(0, RPT // L, reduce, 0)
    pltpu.sync_copy(red, out_hbm.at[c, pl.ds(base, RPT)])


@functools.partial(
    pl.kernel,
    out_type=jax.ShapeDtypeStruct((NC * NPAD, OUT), jnp.float32),
    mesh=_mesh,
    scratch_types=(
        [pltpu.VMEM((CHUNK,), jnp.int32)] * 2
        + [pltpu.VMEM((CHUNK,), jnp.int32)] * 2
        + [pltpu.VMEM((CHUNK, OUT), jnp.float32)]
        + [pltpu.VMEM_SHARED((NPAD, OUT), jnp.float32)]
        + [pltpu.SemaphoreType.DMA] * 3
    ),
)
def _spmm_kernel(hs_hbm, src2_hbm, dst_hbm, z_hbm, out_hbm, *refs):
    isrc = refs[0:2]
    idst = refs[2:4]
    rows, acc, sem = refs[4], refs[5], refs[6]
    isem = refs[7:9]

    c = lax.axis_index("c")
    s = lax.axis_index("s")
    base = s * RPT
    pltpu.sync_copy(z_hbm.at[pl.ds(base, RPT), :], acc.at[pl.ds(base, RPT), :])
    plsc.subcore_barrier()

    e0 = s * EPT

    def fire_idx(j, b):
        off = e0 + j * CHUNK
        pltpu.async_copy(src2_hbm.at[c, pl.ds(off, CHUNK)], isrc[b], isem[b])
        pltpu.async_copy(dst_hbm.at[pl.ds(off, CHUNK)], idst[b], isem[b])

    for b in range(2):
        fire_idx(b, b)

    def step(g, _):
        for b in range(2):
            j = 2 * g + b
            pltpu.make_async_copy(src2_hbm.at[c, pl.ds(0, CHUNK)],
                                  isrc[b], isem[b]).wait()
            pltpu.make_async_copy(dst_hbm.at[pl.ds(0, CHUNK)],
                                  idst[b], isem[b]).wait()
            pltpu.async_copy(hs_hbm.at[isrc[b]], rows, sem).wait()

            @pl.when(j + 2 < NCHUNK)
            def _prefetch():
                fire_idx(j + 2, b)

            pltpu.sync_copy(rows, acc.at[idst[b]], add=True)
        return 0

    lax.fori_loop(0, NCHUNK // 2, step, 0)
    plsc.subcore_barrier()
    pltpu.sync_copy(acc.at[pl.ds(base, RPT), :],
                    out_hbm.at[pl.ds(c * NPAD + base, RPT), :])


def _dis(d0_ref, d1_ref):
    deg = d0_ref[0, 0, :] + d1_ref[0, 0, :] + 1.0
    return lax.rsqrt(deg)


def _stage1_body(x_ref, w_ref, d0_ref, d1_ref, o_ref):
    dis = _dis(d0_ref, d1_ref)
    h = jnp.dot(x_ref[...], w_ref[...], preferred_element_type=jnp.float32)
    o_ref[...] = h * dis[:, None]


_stage1 = pl.pallas_call(
    _stage1_body,
    grid=(NC, GROWS),
    in_specs=[
        pl.BlockSpec((RBLK, IN_CH), lambda c, j: (j, 0)),
        pl.BlockSpec((IN_CH, OUT), lambda c, j: (0, c)),
        pl.BlockSpec((1, 1, RBLK), lambda c, j: (j, 0, 0)),
        pl.BlockSpec((1, 1, RBLK), lambda c, j: (j, 0, 0)),
    ],
    out_specs=pl.BlockSpec((RBLK, OUT), lambda c, j: (c * GROWS + j, 0)),
    out_shape=jax.ShapeDtypeStruct((NC * NPAD, OUT), jnp.float32),
)


def _stage2_body(a0_ref, a1_ref, h0_ref, h1_ref, d0_ref, d1_ref, b_ref,
                 w_ref, o_ref):
    dis = _dis(d0_ref, d1_ref)
    pre0 = (a0_ref[...] + h0_ref[...]) * dis[:, None]
    pre1 = (a1_ref[...] + h1_ref[...]) * dis[:, None]
    h = jnp.concatenate([pre0, pre1], axis=1) + b_ref[0, :][None, :]
    h = jnp.maximum(h, 0.0)
    hc = jnp.dot(h, w_ref[...], preferred_element_type=jnp.float32)
    o_ref[...] = hc * dis[:, None]


_stage2 = pl.pallas_call(
    _stage2_body,
    grid=(NC, GROWS),
    in_specs=[
        pl.BlockSpec((RBLK, OUT), lambda c, j: (j, 0)),
        pl.BlockSpec((RBLK, OUT), lambda c, j: (GROWS + j, 0)),
        pl.BlockSpec((RBLK, OUT), lambda c, j: (j, 0)),
        pl.BlockSpec((RBLK, OUT), lambda c, j: (GROWS + j, 0)),
        pl.BlockSpec((1, 1, RBLK), lambda c, j: (j, 0, 0)),
        pl.BlockSpec((1, 1, RBLK), lambda c, j: (j, 0, 0)),
        pl.BlockSpec((1, HID), lambda c, j: (0, 0)),
        pl.BlockSpec((HID, OUT), lambda c, j: (0, c)),
    ],
    out_specs=pl.BlockSpec((RBLK, OUT), lambda c, j: (c * GROWS + j, 0)),
    out_shape=jax.ShapeDtypeStruct((NC * NPAD, OUT), jnp.float32),
)


def _stage3_body(a0_ref, a1_ref, h0_ref, h1_ref, d0_ref, d1_ref,
                 bmu_ref, bls_ref, mu_ref, ls_ref):
    dis = _dis(d0_ref, d1_ref)
    mu_ref[...] = (a0_ref[...] + h0_ref[...]) * dis[:, None] + bmu_ref[0, :][None, :]
    ls_ref[...] = (a1_ref[...] + h1_ref[...]) * dis[:, None] + bls_ref[0, :][None, :]


_stage3 = pl.pallas_call(
    _stage3_body,
    grid=(GROWS,),
    in_specs=[
        pl.BlockSpec((RBLK, OUT), lambda j: (j, 0)),
        pl.BlockSpec((RBLK, OUT), lambda j: (GROWS + j, 0)),
        pl.BlockSpec((RBLK, OUT), lambda j: (j, 0)),
        pl.BlockSpec((RBLK, OUT), lambda j: (GROWS + j, 0)),
        pl.BlockSpec((1, 1, RBLK), lambda j: (j, 0, 0)),
        pl.BlockSpec((1, 1, RBLK), lambda j: (j, 0, 0)),
        pl.BlockSpec((1, OUT), lambda j: (0, 0)),
        pl.BlockSpec((1, OUT), lambda j: (0, 0)),
    ],
    out_specs=[
        pl.BlockSpec((RBLK, OUT), lambda j: (j, 0)),
        pl.BlockSpec((RBLK, OUT), lambda j: (j, 0)),
    ],
    out_shape=[
        jax.ShapeDtypeStruct((NPAD, OUT), jnp.float32),
        jax.ShapeDtypeStruct((NPAD, OUT), jnp.float32),
    ],
)


def kernel(x, edge_index, W1, b1, Wmu, bmu, Wls, bls):
    src = edge_index[0].astype(jnp.int32)
    dst = edge_index[1].astype(jnp.int32)
    pad = EPAD - N_EDGES
    srcp = jnp.concatenate([src, jnp.zeros((pad,), jnp.int32)])
    dstp = jnp.concatenate([dst, jnp.full((pad,), N_NODES, jnp.int32)])
    src2 = jnp.stack([srcp, srcp + NPAD])
    xp = jnp.pad(x, ((0, NPAD - N_NODES), (0, 0)))
    z = jnp.zeros((NPAD, OUT), jnp.float32)
    Wcat = jnp.concatenate([Wmu, Wls], axis=1)

    degp = _deg_kernel(dstp)
    d0 = degp[0].reshape(GROWS, 1, RBLK)
    d1 = degp[1].reshape(GROWS, 1, RBLK)

    hs1 = _stage1(xp, W1, d0, d1)
    agg1 = _spmm_kernel(hs1, src2, dstp, z)
    hs2 = _stage2(agg1, agg1, hs1, hs1, d0, d1, b1.reshape(1, HID), Wcat)
    agg2 = _spmm_kernel(hs2, src2, dstp, z)
    mu, ls = _stage3(agg2, agg2, hs2, hs2, d0, d1,
                     bmu.reshape(1, OUT), bls.reshape(1, OUT))
    return mu[:N_NODES], ls[:N_NODES]


# speedup vs baseline: 1.5783x; 1.1151x over previous
import functools

import jax
import jax.numpy as jnp
from jax import lax
from jax.experimental import pallas as pl
from jax.experimental.pallas import tpu as pltpu
from jax.experimental.pallas import tpu_sc as plsc

N_NODES = 10000
IN_CH = 256
HID = 256
OUT = 128
N_EDGES = 160000

NC, NS, L = 2, 16, 16
NW = NC * NS

NPAD = 10240
RPT = NPAD // NS
EPAD = 163840
CHUNK = 128
EPT = EPAD // NS
EPW = EPAD // NW
NCHUNK = EPT // CHUNK

RBLK = 1024
GROWS = NPAD // RBLK

_mesh = plsc.VectorSubcoreMesh(core_axis_name="c", subcore_axis_name="s")


@functools.partial(
    pl.kernel,
    out_type=jax.ShapeDtypeStruct((NC, NPAD), jnp.float32),
    mesh=_mesh,
    compiler_params=pltpu.CompilerParams(needs_layout_passes=False),
    scratch_types=[
        pltpu.VMEM((EPW,), jnp.int32),
        pltpu.VMEM((NPAD,), jnp.float32),
        pltpu.VMEM((NS, RPT), jnp.float32),
        pltpu.VMEM((RPT,), jnp.float32),
        pltpu.VMEM_SHARED((NS, NPAD), jnp.float32),
    ],
)
def _deg_kernel(dst_hbm, out_hbm, dbuf, hist, red2, red, sh):
    c = lax.axis_index("c")
    s = lax.axis_index("s")
    g = c * NS + s

    def zero(i, _):
        hist[pl.ds(i * L, L)] = jnp.zeros((L,), jnp.float32)
        return 0

    lax.fori_loop(0, NPAD // L, zero, 0)

    pltpu.sync_copy(dst_hbm.at[pl.ds(g * EPW, EPW)], dbuf)
    ones = jnp.ones((L,), jnp.float32)

    def scat(i, _):
        idx = dbuf[pl.ds(i * L, L)]
        plsc.addupdate_scatter(hist, [idx], ones)
        return 0

    lax.fori_loop(0, EPW // L, scat, 0)

    pltpu.sync_copy(hist, sh.at[s])
    plsc.subcore_barrier()

    base = s * RPT
    pltpu.sync_copy(sh.at[:, pl.ds(base, RPT)], red2)

    def reduce(i, _):
        a = red2[0, pl.ds(i * L, L)]
        for t in range(1, NS):
            a = a + red2[t, pl.ds(i * L, L)]
        red[pl.ds(i * L, L)] = a
        return 0

    lax.fori_loop(0, RPT // L, reduce, 0)
    pltpu.sync_copy(red, out_hbm.at[c, pl.ds(base, RPT)])


@functools.partial(
    pl.kernel,
    out_type=jax.ShapeDtypeStruct((NC * NPAD, OUT), jnp.float32),
    mesh=_mesh,
    scratch_types=(
        [pltpu.VMEM((CHUNK,), jnp.int32)] * 2
        + [pltpu.VMEM((CHUNK,), jnp.int32)] * 2
        + [pltpu.VMEM((CHUNK,), jnp.int32)] * 2
        + [pltpu.VMEM((CHUNK, OUT), jnp.float32)] * 2
        + [pltpu.VMEM_SHARED((NPAD, OUT), jnp.float32)]
        + [pltpu.SemaphoreType.DMA] * 5
    ),
)
def _spmm_kernel(hs_hbm, src2_hbm, dst_hbm, z_hbm, out_hbm, *refs):
    isrc = refs[0:2]
    idst = refs[2:4]
    sdst = refs[4:6]
    rows = refs[6:8]
    acc, sem = refs[8], refs[9]
    isem = refs[10:12]
    ssem = refs[12:14]

    c = lax.axis_index("c")
    s = lax.axis_index("s")
    base = s * RPT
    pltpu.sync_copy(z_hbm.at[pl.ds(base, RPT), :], acc.at[pl.ds(base, RPT), :])
    plsc.subcore_barrier()

    e0 = s * EPT

    def fire_idx(j, b):
        off = e0 + j * CHUNK
        pltpu.async_copy(src2_hbm.at[c, pl.ds(off, CHUNK)], isrc[b], isem[b])
        pltpu.async_copy(dst_hbm.at[pl.ds(off, CHUNK)], idst[b], isem[b])

    for b in range(2):
        fire_idx(b, b)

    def step(g, _):
        for b in range(2):
            j = 2 * g + b
            pltpu.make_async_copy(src2_hbm.at[c, pl.ds(0, CHUNK)],
                                  isrc[b], isem[b]).wait()
            pltpu.make_async_copy(dst_hbm.at[pl.ds(0, CHUNK)],
                                  idst[b], isem[b]).wait()

            @pl.when(g > 0)
            def _drain():
                pltpu.make_async_copy(z_hbm.at[pl.ds(0, CHUNK), :],
                                      rows[b], ssem[b]).wait()

            for k in range(CHUNK // L):
                sdst[b][pl.ds(k * L, L)] = idst[b][pl.ds(k * L, L)]
            pltpu.async_copy(hs_hbm.at[isrc[b]], rows[b], sem).wait()

            @pl.when(j + 2 < NCHUNK)
            def _prefetch():
                fire_idx(j + 2, b)

            pltpu.async_copy(rows[b], acc.at[sdst[b]], ssem[b], add=True)
        return 0

    lax.fori_loop(0, NCHUNK // 2, step, 0)
    for b in range(2):
        pltpu.make_async_copy(z_hbm.at[pl.ds(0, CHUNK), :],
                              rows[b], ssem[b]).wait()
    plsc.subcore_barrier()
    pltpu.sync_copy(acc.at[pl.ds(base, RPT), :],
                    out_hbm.at[pl.ds(c * NPAD + base, RPT), :])


def _dis(d0_ref, d1_ref):
    deg = d0_ref[0, 0, :] + d1_ref[0, 0, :] + 1.0
    return lax.rsqrt(deg)


def _stage1_body(x_ref, w_ref, d0_ref, d1_ref, o_ref):
    dis = _dis(d0_ref, d1_ref)
    h = jnp.dot(x_ref[...], w_ref[...], preferred_element_type=jnp.float32)
    o_ref[...] = h * dis[:, None]


_stage1 = pl.pallas_call(
    _stage1_body,
    grid=(NC, GROWS),
    in_specs=[
        pl.BlockSpec((RBLK, IN_CH), lambda c, j: (j, 0)),
        pl.BlockSpec((IN_CH, OUT), lambda c, j: (0, c)),
        pl.BlockSpec((1, 1, RBLK), lambda c, j: (j, 0, 0)),
        pl.BlockSpec((1, 1, RBLK), lambda c, j: (j, 0, 0)),
    ],
    out_specs=pl.BlockSpec((RBLK, OUT), lambda c, j: (c * GROWS + j, 0)),
    out_shape=jax.ShapeDtypeStruct((NC * NPAD, OUT), jnp.float32),
)


def _stage2_body(a0_ref, a1_ref, h0_ref, h1_ref, d0_ref, d1_ref, b_ref,
                 w_ref, o_ref):
    dis = _dis(d0_ref, d1_ref)
    pre0 = (a0_ref[...] + h0_ref[...]) * dis[:, None]
    pre1 = (a1_ref[...] + h1_ref[...]) * dis[:, None]
    h = jnp.concatenate([pre0, pre1], axis=1) + b_ref[0, :][None, :]
    h = jnp.maximum(h, 0.0)
    hc = jnp.dot(h, w_ref[...], preferred_element_type=jnp.float32)
    o_ref[...] = hc * dis[:, None]


_stage2 = pl.pallas_call(
    _stage2_body,
    grid=(NC, GROWS),
    in_specs=[
        pl.BlockSpec((RBLK, OUT), lambda c, j: (j, 0)),
        pl.BlockSpec((RBLK, OUT), lambda c, j: (GROWS + j, 0)),
        pl.BlockSpec((RBLK, OUT), lambda c, j: (j, 0)),
        pl.BlockSpec((RBLK, OUT), lambda c, j: (GROWS + j, 0)),
        pl.BlockSpec((1, 1, RBLK), lambda c, j: (j, 0, 0)),
        pl.BlockSpec((1, 1, RBLK), lambda c, j: (j, 0, 0)),
        pl.BlockSpec((1, HID), lambda c, j: (0, 0)),
        pl.BlockSpec((HID, OUT), lambda c, j: (0, c)),
    ],
    out_specs=pl.BlockSpec((RBLK, OUT), lambda c, j: (c * GROWS + j, 0)),
    out_shape=jax.ShapeDtypeStruct((NC * NPAD, OUT), jnp.float32),
)


def _stage3_body(a0_ref, a1_ref, h0_ref, h1_ref, d0_ref, d1_ref,
                 bmu_ref, bls_ref, mu_ref, ls_ref):
    dis = _dis(d0_ref, d1_ref)
    mu_ref[...] = (a0_ref[...] + h0_ref[...]) * dis[:, None] + bmu_ref[0, :][None, :]
    ls_ref[...] = (a1_ref[...] + h1_ref[...]) * dis[:, None] + bls_ref[0, :][None, :]


_stage3 = pl.pallas_call(
    _stage3_body,
    grid=(GROWS,),
    in_specs=[
        pl.BlockSpec((RBLK, OUT), lambda j: (j, 0)),
        pl.BlockSpec((RBLK, OUT), lambda j: (GROWS + j, 0)),
        pl.BlockSpec((RBLK, OUT), lambda j: (j, 0)),
        pl.BlockSpec((RBLK, OUT), lambda j: (GROWS + j, 0)),
        pl.BlockSpec((1, 1, RBLK), lambda j: (j, 0, 0)),
        pl.BlockSpec((1, 1, RBLK), lambda j: (j, 0, 0)),
        pl.BlockSpec((1, OUT), lambda j: (0, 0)),
        pl.BlockSpec((1, OUT), lambda j: (0, 0)),
    ],
    out_specs=[
        pl.BlockSpec((RBLK, OUT), lambda j: (j, 0)),
        pl.BlockSpec((RBLK, OUT), lambda j: (j, 0)),
    ],
    out_shape=[
        jax.ShapeDtypeStruct((NPAD, OUT), jnp.float32),
        jax.ShapeDtypeStruct((NPAD, OUT), jnp.float32),
    ],
)


def kernel(x, edge_index, W1, b1, Wmu, bmu, Wls, bls):
    src = edge_index[0].astype(jnp.int32)
    dst = edge_index[1].astype(jnp.int32)
    pad = EPAD - N_EDGES
    srcp = jnp.concatenate([src, jnp.zeros((pad,), jnp.int32)])
    dstp = jnp.concatenate([dst, jnp.full((pad,), N_NODES, jnp.int32)])
    src2 = jnp.stack([srcp, srcp + NPAD])
    xp = jnp.pad(x, ((0, NPAD - N_NODES), (0, 0)))
    z = jnp.zeros((NPAD, OUT), jnp.float32)
    Wcat = jnp.concatenate([Wmu, Wls], axis=1)

    degp = _deg_kernel(dstp)
    d0 = degp[0].reshape(GROWS, 1, RBLK)
    d1 = degp[1].reshape(GROWS, 1, RBLK)

    hs1 = _stage1(xp, W1, d0, d1)
    agg1 = _spmm_kernel(hs1, src2, dstp, z)
    hs2 = _stage2(agg1, agg1, hs1, hs1, d0, d1, b1.reshape(1, HID), Wcat)
    agg2 = _spmm_kernel(hs2, src2, dstp, z)
    mu, ls = _stage3(agg2, agg2, hs2, hs2, d0, d1,
                     bmu.reshape(1, OUT), bls.reshape(1, OUT))
    return mu[:N_NODES], ls[:N_NODES]
